# Initial kernel scaffold; baseline (speedup 1.0000x reference)
import jax
import jax.numpy as jnp
from jax.experimental import pallas as pl


def kernel(x, Wg, W1, W2, W3, Ws1, Ws2, Ws3, routing_bias):
    raise NotImplementedError("write your pallas kernel here")


# speedup vs baseline: 1.0504x; 1.0504x over previous
import functools

import jax
import jax.numpy as jnp
from jax.experimental import pallas as pl
from jax.experimental.pallas import tpu as pltpu

E = 8
TOP_K = 4


def _moe_body(x_ref, wgt_ref, bias_ref, wa_ref, wb_ref, wc_ref,
              out_ref, coef_ref, *, T):
    j = pl.program_id(1)

    xf = x_ref[...]
    xb = xf.astype(jnp.bfloat16)

    @pl.when(j == 0)
    def _():
        g = jax.lax.dot_general(
            xb, wgt_ref[...], (((1,), (0,)), ((), ())),
            preferred_element_type=jnp.float32)
        g = g + bias_ref[...]
        lane = jax.lax.broadcasted_iota(jnp.int32, (T, E), 1)
        cnt = jnp.zeros((T, E), jnp.int32)
        for jj in range(E):
            gj = g[:, jj:jj + 1]
            above = (gj > g) | ((gj == g) & (jj < lane))
            cnt = cnt + above.astype(jnp.int32)
        sel = cnt < TOP_K
        m = jnp.max(g, axis=1, keepdims=True)
        p = jnp.where(sel, jnp.exp(g - m), 0.0)
        coef_ref[...] = p / jnp.sum(p, axis=1, keepdims=True)

    h = jax.lax.dot_general(xb, wa_ref[0], (((1,), (0,)), ((), ())),
                            preferred_element_type=jnp.float32)
    h = h * jax.nn.sigmoid(h)
    v = jax.lax.dot_general(xb, wb_ref[0], (((1,), (0,)), ((), ())),
                            preferred_element_type=jnp.float32)
    hv = (h * v).astype(jnp.bfloat16)
    y = jax.lax.dot_general(hv, wc_ref[0], (((1,), (0,)), ((), ())),
                            preferred_element_type=jnp.float32)

    @pl.when(j == 0)
    def _():
        out_ref[...] = y

    @pl.when(j > 0)
    def _():
        lane = jax.lax.broadcasted_iota(jnp.int32, (T, E), 1)
        c = jnp.sum(jnp.where(lane == (j - 1), coef_ref[...], 0.0),
                    axis=1, keepdims=True)
        out_ref[...] += y * c


def kernel(x, Wg, W1, W2, W3, Ws1, Ws2, Ws3, routing_bias):
    B, S, D = x.shape
    En, H, _ = W1.shape
    N = B * S
    T = min(1024, N)
    xf = x.reshape(N, D)

    Wa = jnp.concatenate([Ws1[None], W1], 0).swapaxes(1, 2).astype(jnp.bfloat16)
    Wb = jnp.concatenate([Ws2[None], W2], 0).swapaxes(1, 2).astype(jnp.bfloat16)
    Wc = jnp.concatenate([Ws3[None], W3], 0).swapaxes(1, 2).astype(jnp.bfloat16)
    WgT = Wg.T.astype(jnp.bfloat16)
    bias = routing_bias.reshape(1, En).astype(jnp.float32)

    grid = (N // T, En + 1)
    out = pl.pallas_call(
        functools.partial(_moe_body, T=T),
        grid=grid,
        in_specs=[
            pl.BlockSpec((T, D), lambda tb, j: (tb, 0)),
            pl.BlockSpec((D, En), lambda tb, j: (0, 0)),
            pl.BlockSpec((1, En), lambda tb, j: (0, 0)),
            pl.BlockSpec((1, D, H), lambda tb, j: (j, 0, 0)),
            pl.BlockSpec((1, D, H), lambda tb, j: (j, 0, 0)),
            pl.BlockSpec((1, H, D), lambda tb, j: (j, 0, 0)),
        ],
        out_specs=pl.BlockSpec((T, D), lambda tb, j: (tb, 0)),
        out_shape=jax.ShapeDtypeStruct((N, D), jnp.float32),
        scratch_shapes=[pltpu.VMEM((T, En), jnp.float32)],
        compiler_params=pltpu.CompilerParams(
            dimension_semantics=("arbitrary", "arbitrary")),
    )(xf, WgT, bias, Wa, Wb, Wc)
    return out.reshape(B, S, D)
